# Initial kernel scaffold; baseline (speedup 1.0000x reference)
import jax
import jax.numpy as jnp
from jax.experimental import pallas as pl


def kernel(x, edge_index):
    raise NotImplementedError("write your pallas kernel here")


# speedup vs baseline: 2.8654x; 2.8654x over previous
import functools

import jax
import jax.numpy as jnp
from jax import lax
from jax.experimental import pallas as pl
from jax.experimental.pallas import tpu as pltpu
from jax.experimental.pallas import tpu_sc as plsc

V = 10000
D = 128
B = 320000

_info = plsc.get_sparse_core_info()
NC, NS = _info.num_cores, _info.num_subcores
NW = NC * NS
B_PER_W = B // NW
C = 80
N_CHUNKS = B_PER_W // C

_mesh = plsc.VectorSubcoreMesh(core_axis_name="c", subcore_axis_name="s")


@functools.partial(
    pl.kernel,
    mesh=_mesh,
    out_type=jax.ShapeDtypeStruct((B, D), jnp.float32),
    scratch_types=[
        pltpu.VMEM((C,), jnp.int32),
        pltpu.VMEM((C, D), jnp.float32),
        pltpu.SemaphoreType.DMA,
    ],
)
def _gather_sc(x_hbm, idx_hbm, out_hbm, idx_v, rows_v, sem):
    wid = lax.axis_index("s") * NC + lax.axis_index("c")
    base_w = wid * B_PER_W

    def body(i, carry):
        base = base_w + i * C
        pltpu.sync_copy(idx_hbm.at[pl.ds(base, C)], idx_v)
        pltpu.async_copy(x_hbm.at[idx_v], rows_v, sem).wait()
        pltpu.sync_copy(rows_v, out_hbm.at[pl.ds(base, C)])
        return carry

    lax.fori_loop(0, N_CHUNKS, body, 0)


def kernel(x, edge_index):
    idx = edge_index[1]
    return _gather_sc(x, idx)


# speedup vs baseline: 5.5425x; 1.9343x over previous
import functools

import jax
import jax.numpy as jnp
from jax import lax
from jax.experimental import pallas as pl
from jax.experimental.pallas import tpu as pltpu
from jax.experimental.pallas import tpu_sc as plsc

V = 10000
D = 128
B = 320000

_info = plsc.get_sparse_core_info()
NC, NS = _info.num_cores, _info.num_subcores
NW = NC * NS
B_PER_W = B // NW
C = 80
N_CHUNKS = B_PER_W // C
NBUF = 5
G = N_CHUNKS // NBUF

_mesh = plsc.VectorSubcoreMesh(core_axis_name="c", subcore_axis_name="s")


@functools.partial(
    pl.kernel,
    mesh=_mesh,
    out_type=jax.ShapeDtypeStruct((B, D), jnp.float32),
    scratch_types=[
        pltpu.VMEM((B_PER_W,), jnp.int32),
        pltpu.VMEM((NBUF, C, D), jnp.float32),
        pltpu.SemaphoreType.DMA((NBUF,)),
        pltpu.SemaphoreType.DMA((NBUF,)),
    ],
)
def _gather_sc(x_hbm, idx_hbm, out_hbm, idx_v, rows_v, gsem, ssem):
    wid = lax.axis_index("s") * NC + lax.axis_index("c")
    base_w = wid * B_PER_W

    pltpu.sync_copy(idx_hbm.at[pl.ds(base_w, B_PER_W)], idx_v)

    def gather_copy(i, b):
        off = pl.multiple_of(i * C, 8)
        return pltpu.make_async_copy(
            x_hbm.at[idx_v.at[pl.ds(off, C)]], rows_v.at[b], gsem.at[b])

    def store_copy(i, b):
        return pltpu.make_async_copy(
            rows_v.at[b], out_hbm.at[pl.ds(base_w + i * C, C)], ssem.at[b])

    def outer(g, carry):
        for b in range(NBUF):
            i = g * NBUF + b
            @pl.when(g > 0)
            def _():
                store_copy(0, b).wait()

            gather_copy(i, b).start()

            pb = (b - 1) % NBUF
            if b == 0:
                @pl.when(g > 0)
                def _():
                    gather_copy(0, pb).wait()
                    store_copy(g * NBUF - 1, pb).start()
            else:
                gather_copy(0, pb).wait()
                store_copy(i - 1, pb).start()
        return carry

    lax.fori_loop(0, G, outer, 0)

    last = N_CHUNKS - 1
    lb = last % NBUF
    gather_copy(0, lb).wait()
    store_copy(last, lb).start()
    for b in range(NBUF):
        store_copy(0, b).wait()


def kernel(x, edge_index):
    return _gather_sc(x, edge_index[1])


# speedup vs baseline: 5.7533x; 1.0380x over previous
import functools

import jax
import jax.numpy as jnp
from jax import lax
from jax.experimental import pallas as pl
from jax.experimental.pallas import tpu as pltpu
from jax.experimental.pallas import tpu_sc as plsc

V = 10000
D = 128
B = 320000

_info = plsc.get_sparse_core_info()
NC, NS = _info.num_cores, _info.num_subcores
NW = NC * NS
B_PER_W = B // NW
C = 80
N_CHUNKS = B_PER_W // C
NBUF = 5
G = N_CHUNKS // NBUF

_mesh = plsc.VectorSubcoreMesh(core_axis_name="c", subcore_axis_name="s")


@functools.partial(
    pl.kernel,
    mesh=_mesh,
    out_type=jax.ShapeDtypeStruct((B, D), jnp.float32),
    scratch_types=[
        pltpu.VMEM((B_PER_W,), jnp.int32),
        pltpu.VMEM((NBUF, C, D), jnp.float32),
        pltpu.SemaphoreType.DMA((NBUF,)),
        pltpu.SemaphoreType.DMA((NBUF,)),
    ],
)
def _gather_sc(x_hbm, idx_hbm, out_hbm, idx_v, rows_v, gsem, ssem):
    wid = lax.axis_index("s") * NC + lax.axis_index("c")
    base_w = wid * B_PER_W

    pltpu.sync_copy(idx_hbm.at[pl.ds(base_w, B_PER_W)], idx_v)

    def gather_copy(i, b):
        off = pl.multiple_of(i * C, 8)
        return pltpu.make_async_copy(
            x_hbm.at[idx_v.at[pl.ds(off, C)]], rows_v.at[b], gsem.at[b])

    def store_copy(i, b):
        return pltpu.make_async_copy(
            rows_v.at[b], out_hbm.at[pl.ds(base_w + i * C, C)], ssem.at[b])

    SKEW = 2

    def outer(g, carry):
        for b in range(NBUF):
            i = g * NBUF + b
            @pl.when(g > 0)
            def _():
                store_copy(0, b).wait()

            gather_copy(i, b).start()

            pb = (b - SKEW) % NBUF
            if b >= SKEW:
                gather_copy(0, pb).wait()
                store_copy(i - SKEW, pb).start()
            else:
                @pl.when(g > 0)
                def _():
                    gather_copy(0, pb).wait()
                    store_copy(g * NBUF + b - SKEW, pb).start()
        return carry

    lax.fori_loop(0, G, outer, 0)

    for k in range(SKEW, 0, -1):
        last = N_CHUNKS - k
        lb = last % NBUF
        gather_copy(0, lb).wait()
        store_copy(last, lb).start()
    for b in range(NBUF):
        store_copy(0, b).wait()


def kernel(x, edge_index):
    return _gather_sc(x, edge_index[1])


# speedup vs baseline: 8.5502x; 1.4861x over previous
import functools

import jax
import jax.numpy as jnp
from jax import lax
from jax.experimental import pallas as pl
from jax.experimental.pallas import tpu as pltpu
from jax.experimental.pallas import tpu_sc as plsc

V = 10000
D = 128
B = 320000

_info = plsc.get_sparse_core_info()
NC, NS = _info.num_cores, _info.num_subcores
NW = NC * NS
B_PER_W = B // NW
C = 80
N_CHUNKS = B_PER_W // C
NBUF = 4
G = (N_CHUNKS - 1) // NBUF

_mesh = plsc.VectorSubcoreMesh(core_axis_name="c", subcore_axis_name="s")


@functools.partial(
    pl.kernel,
    mesh=_mesh,
    out_type=jax.ShapeDtypeStruct((B, D), jnp.float32),
    scratch_types=[
        pltpu.VMEM((NBUF, C), jnp.int32),
        pltpu.VMEM((NBUF, C, D), jnp.float32),
        pltpu.VMEM_SHARED((V, D), jnp.float32),
        pltpu.SemaphoreType.DMA((NBUF,)),
        pltpu.SemaphoreType.DMA((NBUF,)),
        pltpu.SemaphoreType.DMA((NBUF,)),
    ],
)
def _gather_sc(x_hbm, idx_hbm, out_hbm, idx_v, rows_v, xs, isem, gsem, ssem):
    sid = lax.axis_index("s")
    wid = sid * NC + lax.axis_index("c")
    base_w = wid * B_PER_W

    RS = 632
    @pl.when(sid < NS - 1)
    def _():
        r0 = pl.multiple_of(sid * RS, 8)
        pltpu.sync_copy(x_hbm.at[pl.ds(r0, RS)], xs.at[pl.ds(r0, RS)])

    @pl.when(sid == NS - 1)
    def _():
        r0 = (NS - 1) * RS
        pltpu.sync_copy(x_hbm.at[pl.ds(r0, V - r0)], xs.at[pl.ds(r0, V - r0)])

    def idx_copy(i, b):
        off = pl.multiple_of(base_w + i * C, 8)
        return pltpu.make_async_copy(
            idx_hbm.at[pl.ds(off, C)], idx_v.at[b], isem.at[b])

    def gather_copy(b):
        return pltpu.make_async_copy(
            xs.at[idx_v.at[b]], rows_v.at[b], gsem.at[b])

    def store_copy(i, b):
        off = pl.multiple_of(base_w + i * C, 8)
        return pltpu.make_async_copy(
            rows_v.at[b], out_hbm.at[pl.ds(off, C)], ssem.at[b])

    idx_copy(0, 0).start()
    idx_copy(1, 1).start()
    plsc.subcore_barrier()

    SKEW = 2

    def outer(g, carry):
        for b in range(NBUF):
            i = g * NBUF + b
            @pl.when(g > 0)
            def _():
                store_copy(0, b).wait()

            pb = (b - SKEW) % NBUF
            if b >= SKEW:
                gather_copy(pb).wait()
                store_copy(i - SKEW, pb).start()
            else:
                @pl.when(g > 0)
                def _():
                    gather_copy(pb).wait()
                    store_copy(g * NBUF + b - SKEW, pb).start()

            @pl.when(i <= N_CHUNKS - 1 - SKEW)
            def _():
                idx_copy(i + SKEW, (b + SKEW) % NBUF).start()

            idx_copy(0, b).wait()
            gather_copy(b).start()
        return carry

    lax.fori_loop(0, G, outer, 0)

    gather_copy(2).wait()
    store_copy(N_CHUNKS - 3, 2).start()
    store_copy(0, 0).wait()
    idx_copy(0, 0).wait()
    gather_copy(0).start()
    gather_copy(3).wait()
    store_copy(N_CHUNKS - 2, 3).start()
    gather_copy(0).wait()
    store_copy(N_CHUNKS - 1, 0).start()
    for b in range(1, NBUF):
        store_copy(0, b).wait()
    store_copy(0, 0).wait()


def kernel(x, edge_index):
    return _gather_sc(x, edge_index[1])


# speedup vs baseline: 9.5189x; 1.1133x over previous
import functools

import jax
import jax.numpy as jnp
from jax import lax
from jax.experimental import pallas as pl
from jax.experimental.pallas import tpu as pltpu
from jax.experimental.pallas import tpu_sc as plsc

V = 10000
D = 128
B = 320000

_info = plsc.get_sparse_core_info()
NC, NS = _info.num_cores, _info.num_subcores
NW = NC * NS
B_PER_W = B // NW
C = 80
N_CHUNKS = B_PER_W // C
NBUF = 4
G = (N_CHUNKS - 1) // NBUF

_mesh = plsc.VectorSubcoreMesh(core_axis_name="c", subcore_axis_name="s")


@functools.partial(
    pl.kernel,
    mesh=_mesh,
    out_type=jax.ShapeDtypeStruct((B, D), jnp.float32),
    scratch_types=[
        pltpu.VMEM((NBUF, C), jnp.int32),
        pltpu.VMEM((NBUF, C, D), jnp.float32),
        pltpu.VMEM_SHARED((V, D), jnp.float32),
        pltpu.SemaphoreType.DMA((NBUF,)),
        pltpu.SemaphoreType.DMA((NBUF,)),
        pltpu.SemaphoreType.DMA((NBUF,)),
    ],
)
def _gather_sc(x_hbm, idx_hbm, out_hbm, idx_v, rows_v, xs, isem, gsem, ssem):
    sid = lax.axis_index("s")
    wid = sid * NC + lax.axis_index("c")
    base_w = wid * B_PER_W

    RS = 632
    @pl.when(sid < NS - 1)
    def _():
        r0 = pl.multiple_of(sid * RS, 8)
        pltpu.sync_copy(x_hbm.at[pl.ds(r0, RS)], xs.at[pl.ds(r0, RS)])

    @pl.when(sid == NS - 1)
    def _():
        r0 = (NS - 1) * RS
        pltpu.sync_copy(x_hbm.at[pl.ds(r0, V - r0)], xs.at[pl.ds(r0, V - r0)])

    def idx_copy(i, b):
        off = pl.multiple_of(B + base_w + i * C, 8)
        return pltpu.make_async_copy(
            idx_hbm.at[pl.ds(off, C)], idx_v.at[b], isem.at[b])

    def gather_copy(b):
        return pltpu.make_async_copy(
            xs.at[idx_v.at[b]], rows_v.at[b], gsem.at[b])

    def store_copy(i, b):
        off = pl.multiple_of(base_w + i * C, 8)
        return pltpu.make_async_copy(
            rows_v.at[b], out_hbm.at[pl.ds(off, C)], ssem.at[b])

    idx_copy(0, 0).start()
    idx_copy(1, 1).start()
    plsc.subcore_barrier()

    SKEW = 2

    def outer(g, carry):
        for b in range(NBUF):
            i = g * NBUF + b
            @pl.when(g > 0)
            def _():
                store_copy(0, b).wait()

            pb = (b - SKEW) % NBUF
            if b >= SKEW:
                gather_copy(pb).wait()
                store_copy(i - SKEW, pb).start()
            else:
                @pl.when(g > 0)
                def _():
                    gather_copy(pb).wait()
                    store_copy(g * NBUF + b - SKEW, pb).start()

            @pl.when(i <= N_CHUNKS - 1 - SKEW)
            def _():
                idx_copy(i + SKEW, (b + SKEW) % NBUF).start()

            idx_copy(0, b).wait()
            gather_copy(b).start()
        return carry

    lax.fori_loop(0, G, outer, 0)

    gather_copy(2).wait()
    store_copy(N_CHUNKS - 3, 2).start()
    store_copy(0, 0).wait()
    idx_copy(0, 0).wait()
    gather_copy(0).start()
    gather_copy(3).wait()
    store_copy(N_CHUNKS - 2, 3).start()
    gather_copy(0).wait()
    store_copy(N_CHUNKS - 1, 0).start()
    for b in range(1, NBUF):
        store_copy(0, b).wait()
    store_copy(0, 0).wait()


def kernel(x, edge_index):
    return _gather_sc(x, edge_index.reshape(-1))
